# baseline (speedup 1.0000x reference)
import jax
import jax.numpy as jnp
from jax import lax
from jax.experimental import pallas as pl
from jax.experimental.pallas import tpu as pltpu
from jax.experimental.pallas import tpu_sc as plsc

B = 16384
DIM = 32
NI = 5
NC = 2
NS = 16
NW = NC * NS
BPW = B // NW
SB = 128
NSB = BPW // SB
CHUNK = 128
PACK = 128 // DIM
TROWS = 1000000 // PACK


def _sc_body(uid_hbm, ids_hbm, utab_hbm, itab_hbm, s_hbm, d0_hbm,
             uidx_v, iidx_v, ubidx_v, ibidx_v, urows_v, irows_v,
             s_v, d0_v, sem):
    wid = lax.axis_index("s") * NC + lax.axis_index("c")
    base = wid * BPW

    pltpu.sync_copy(uid_hbm.at[pl.ds(base, BPW)], uidx_v)
    pltpu.sync_copy(ids_hbm.at[pl.ds(base * NI, BPW * NI)], iidx_v)
    for v in range(BPW // 16):
        ubidx_v[pl.ds(v * 16, 16)] = jnp.right_shift(uidx_v[pl.ds(v * 16, 16)], 2)
    def shift_items(v, carry):
        ibidx_v[pl.ds(v * 16, 16)] = jnp.right_shift(iidx_v[pl.ds(v * 16, 16)], 2)
        return carry
    lax.fori_loop(0, BPW * NI // 16, shift_items, 0)

    iota16 = lax.broadcasted_iota(jnp.int32, (16,), 0)

    for sb in range(NSB):
        copies = [pltpu.async_copy(
            utab_hbm.at[ubidx_v.at[pl.ds(sb * SB, CHUNK)]], urows_v, sem)]
        for c in range(SB * NI // CHUNK):
            copies.append(pltpu.async_copy(
                itab_hbm.at[ibidx_v.at[pl.ds(sb * SB * NI + c * CHUNK, CHUNK)]],
                irows_v.at[pl.ds(c * CHUNK, CHUNK)], sem))
        for cp in copies:
            cp.wait()

        def group(g, carry):
            lrow = g * 16 + iota16
            grow = sb * SB + g * 16 + iota16
            uids = plsc.load_gather(uidx_v, [grow])
            ucol = (uids & 3) * DIM
            icols = []
            irows = []
            for j in range(NI):
                pos = grow * NI + j
                ids_j = plsc.load_gather(iidx_v, [pos])
                icols.append((ids_j & 3) * DIM)
                irows.append(lrow * NI + j)
            acc = [jnp.zeros((16,), jnp.float32) for _ in range(NI)]
            for d in range(DIM):
                u = plsc.load_gather(urows_v, [lrow, ucol + d])
                for j in range(NI):
                    it = plsc.load_gather(irows_v, [irows[j], icols[j] + d])
                    acc[j] = acc[j] + u * it
            ssum = jnp.exp(acc[0])
            for j in range(1, NI):
                ssum = ssum + jnp.exp(acc[j])
            s_v[pl.ds(sb * SB + g * 16, 16)] = ssum
            d0_v[pl.ds(sb * SB + g * 16, 16)] = acc[0]
            return carry

        lax.fori_loop(0, SB // 16, group, 0)

    pltpu.sync_copy(s_v, s_hbm.at[pl.ds(base, BPW)])
    pltpu.sync_copy(d0_v, d0_hbm.at[pl.ds(base, BPW)])


_sc_call = pl.kernel(
    _sc_body,
    mesh=plsc.VectorSubcoreMesh(core_axis_name="c", subcore_axis_name="s"),
    compiler_params=pltpu.CompilerParams(
        use_tc_tiling_on_sc=True, needs_layout_passes=False),
    out_type=[
        jax.ShapeDtypeStruct((B,), jnp.float32),
        jax.ShapeDtypeStruct((B,), jnp.float32),
    ],
    scratch_types=[
        pltpu.VMEM((BPW,), jnp.int32),
        pltpu.VMEM((BPW * NI,), jnp.int32),
        pltpu.VMEM((BPW,), jnp.int32),
        pltpu.VMEM((BPW * NI,), jnp.int32),
        pltpu.VMEM((SB, 128), jnp.float32),
        pltpu.VMEM((SB * NI, 128), jnp.float32),
        pltpu.VMEM((BPW,), jnp.float32),
        pltpu.VMEM((BPW,), jnp.float32),
        pltpu.SemaphoreType.DMA,
    ],
)


def _tc_loss_body(s_ref, d0_ref, out_ref):
    out_ref[0, 0] = (jnp.sum(jnp.log(s_ref[:])) - jnp.sum(d0_ref[:])) / B


_tc_loss = pl.pallas_call(
    _tc_loss_body,
    out_shape=jax.ShapeDtypeStruct((1, 1), jnp.float32),
    out_specs=pl.BlockSpec(memory_space=pltpu.SMEM),
)


def kernel(userid, itemid, user_feature, item_feature, neg_sample,
           user_table, item_table):
    uid = userid.reshape(B).astype(jnp.int32)
    ids = jnp.concatenate(
        [itemid.astype(jnp.int32), neg_sample.astype(jnp.int32)], axis=1
    ).reshape(B * NI)
    utab = user_table.reshape(TROWS, 128)
    itab = item_table.reshape(TROWS, 128)
    s, d0 = _sc_call(uid, ids, utab, itab)
    loss = _tc_loss(s.reshape(B // 128, 128), d0.reshape(B // 128, 128))
    return loss[0, 0]
